# baseline (speedup 1.0000x reference)
import functools

import jax
import jax.numpy as jnp
from jax import lax
from jax.experimental import pallas as pl
from jax.experimental.pallas import tpu as pltpu
from jax.experimental.pallas import tpu_sc as plsc

_B = 16384
_D = 64
_NEG = 20
_NC = 2
_NS = 16
_NW = _NC * _NS
_BW = _B // _NW
_C = 64
_NCH = _BW // _C
_SEG = _C * (1 + _NEG)
_NSCORE = _B * (1 + _NEG)


def _sc_scores(pos_u, pos_v, neg_r, U, V):
    mesh = plsc.VectorSubcoreMesh(core_axis_name="c", subcore_axis_name="s")

    @functools.partial(
        pl.kernel,
        out_type=jax.ShapeDtypeStruct((_NSCORE,), jnp.float32),
        mesh=mesh,
        compiler_params=pltpu.CompilerParams(needs_layout_passes=False,
                                             use_tc_tiling_on_sc=False),
        scratch_types=[
            pltpu.VMEM((_C,), jnp.int32),
            pltpu.VMEM((_C,), jnp.int32),
            pltpu.VMEM((_NEG * _C,), jnp.int32),
            pltpu.VMEM((_C, _D), jnp.float32),
            pltpu.VMEM((_C, _D), jnp.float32),
            pltpu.VMEM((_NEG * _C, _D), jnp.float32),
            pltpu.VMEM((_C,), jnp.float32),
            pltpu.VMEM((_NEG * _C + 16,), jnp.float32),
            pltpu.SemaphoreType.DMA,
        ],
    )
    def k(pos_u_h, pos_v_h, neg_r_h, U_h, V_h, out_h,
          idx_u, idx_v, idx_n, u_rows, v_rows, n_rows, sc_pos, sc_neg, sem):
        wid = lax.axis_index("s") * _NC + lax.axis_index("c")
        li = lax.broadcasted_iota(jnp.int32, (16,), 0)

        def chunk_body(c, _):
            seg = wid * _NCH + c
            cbase = seg * _C
            pltpu.sync_copy(pos_u_h.at[pl.ds(cbase, _C)], idx_u)
            pltpu.sync_copy(pos_v_h.at[pl.ds(cbase, _C)], idx_v)
            pltpu.sync_copy(neg_r_h.at[pl.ds(seg * _NEG * _C, _NEG * _C)],
                            idx_n)
            cps = [
                pltpu.async_copy(U_h.at[idx_u], u_rows, sem),
                pltpu.async_copy(V_h.at[idx_v], v_rows, sem),
            ]
            for j in range(_NEG):
                cps.append(pltpu.async_copy(
                    V_h.at[idx_n.at[pl.ds(j * _C, _C)]],
                    n_rows.at[pl.ds(j * _C, _C)], sem))
            for cp in cps:
                cp.wait()

            def pos_body(g, _):
                acc = jnp.zeros((16,), jnp.float32)
                for kk in range(16):
                    b = g * 16 + kk
                    p = (u_rows[b, pl.ds(0, 16)] * v_rows[b, pl.ds(0, 16)]
                         + u_rows[b, pl.ds(16, 16)] * v_rows[b, pl.ds(16, 16)]
                         + u_rows[b, pl.ds(32, 16)] * v_rows[b, pl.ds(32, 16)]
                         + u_rows[b, pl.ds(48, 16)] * v_rows[b, pl.ds(48, 16)])
                    acc = jnp.where(li == kk, jnp.sum(p), acc)
                sc_pos[pl.ds(g * 16, 16)] = acc
                return 0

            lax.fori_loop(0, _C // 16, pos_body, 0)

            def neg_body(b, _):
                u0 = u_rows[b, pl.ds(0, 16)]
                u1 = u_rows[b, pl.ds(16, 16)]
                u2 = u_rows[b, pl.ds(32, 16)]
                u3 = u_rows[b, pl.ds(48, 16)]
                acc1 = jnp.zeros((16,), jnp.float32)
                acc2 = jnp.zeros((16,), jnp.float32)
                for j in range(_NEG):
                    r = j * _C + b
                    p = (n_rows[r, pl.ds(0, 16)] * u0
                         + n_rows[r, pl.ds(16, 16)] * u1
                         + n_rows[r, pl.ds(32, 16)] * u2
                         + n_rows[r, pl.ds(48, 16)] * u3)
                    s = -jnp.sum(p)
                    if j < 16:
                        acc1 = jnp.where(li == j, s, acc1)
                    else:
                        acc2 = jnp.where(li == (j - 16), s, acc2)
                sc_neg[pl.ds(b * _NEG, 16)] = acc1
                tail = sc_neg[pl.ds(b * _NEG + 16, 16)]
                sc_neg[pl.ds(b * _NEG + 16, 16)] = jnp.where(li < 4, acc2,
                                                             tail)
                return 0

            lax.fori_loop(0, _C, neg_body, 0)

            obase = seg * _SEG
            pltpu.sync_copy(sc_pos, out_h.at[pl.ds(obase, _C)])
            pltpu.sync_copy(sc_neg.at[pl.ds(0, _NEG * _C)],
                            out_h.at[pl.ds(obase + _C, _NEG * _C)])
            return 0

        lax.fori_loop(0, _NCH, chunk_body, 0)

    return k(pos_u, pos_v, neg_r, U, V)


def _tc_transpose(xt):
    d, n = xt.shape
    bn = 2048

    def body(x_ref, o_ref):
        o_ref[...] = x_ref[...].T

    return pl.pallas_call(
        body,
        grid=(pl.cdiv(n, bn),),
        in_specs=[pl.BlockSpec((d, bn), lambda i: (0, i))],
        out_specs=pl.BlockSpec((bn, d), lambda i: (i, 0)),
        out_shape=jax.ShapeDtypeStruct((n, d), jnp.float32),
    )(xt)


def _tc_logsig_sum(x2d):
    def body(x_ref, o_ref):
        x = x_ref[...]
        ls = jnp.minimum(x, 0.0) - jnp.log1p(jnp.exp(-jnp.abs(x)))
        o_ref[0, 0] = -jnp.sum(ls)

    return pl.pallas_call(
        body,
        out_shape=jax.ShapeDtypeStruct((1, 1), jnp.float32),
        out_specs=pl.BlockSpec(memory_space=pltpu.SMEM),
    )(x2d)


def kernel(pos_u, pos_v, neg_v, U, V):
    neg_r = neg_v.reshape(_NW * _NCH, _C, _NEG).transpose(0, 2, 1).reshape(-1)
    U_rm = _tc_transpose(U.T)
    V_rm = _tc_transpose(V.T)
    scores = _sc_scores(pos_u, pos_v, neg_r, U_rm, V_rm)
    res = _tc_logsig_sum(scores.reshape(_NSCORE // 128, 128))
    return res[0, 0]


# speedup vs baseline: 1.1486x; 1.1486x over previous
import functools

import jax
import jax.numpy as jnp
from jax import lax
from jax.experimental import pallas as pl
from jax.experimental.pallas import tpu as pltpu
from jax.experimental.pallas import tpu_sc as plsc

_B = 16384
_D = 64
_NEG = 20
_NC = 2
_NS = 16
_NW = _NC * _NS
_BW = _B // _NW
_C = 64
_NCH = _BW // _C
_SEG = _C * (1 + _NEG)
_NSCORE = _B * (1 + _NEG)


def _sc_scores(pos_u, pos_v, neg_r, U, V):
    mesh = plsc.VectorSubcoreMesh(core_axis_name="c", subcore_axis_name="s")

    @functools.partial(
        pl.kernel,
        out_type=jax.ShapeDtypeStruct((_NSCORE,), jnp.float32),
        mesh=mesh,
        compiler_params=pltpu.CompilerParams(needs_layout_passes=False,
                                             use_tc_tiling_on_sc=False),
        scratch_types=[
            pltpu.VMEM((_C,), jnp.int32),
            pltpu.VMEM((_C,), jnp.int32),
            pltpu.VMEM((_NEG * _C,), jnp.int32),
            pltpu.VMEM((_C, _D), jnp.float32),
            pltpu.VMEM((_C, _D), jnp.float32),
            pltpu.VMEM((_NEG * _C, _D), jnp.float32),
            pltpu.VMEM((_C,), jnp.float32),
            pltpu.VMEM((_NEG * _C + 16,), jnp.float32),
            pltpu.SemaphoreType.DMA,
        ],
    )
    def k(pos_u_h, pos_v_h, neg_r_h, U_h, V_h, out_h,
          idx_u, idx_v, idx_n, u_rows, v_rows, n_rows, sc_pos, sc_neg, sem):
        wid = lax.axis_index("s") * _NC + lax.axis_index("c")
        li = lax.broadcasted_iota(jnp.int32, (16,), 0)

        def chunk_body(c, _):
            seg = wid * _NCH + c
            cbase = seg * _C
            pltpu.sync_copy(pos_u_h.at[pl.ds(cbase, _C)], idx_u)
            pltpu.sync_copy(pos_v_h.at[pl.ds(cbase, _C)], idx_v)
            pltpu.sync_copy(neg_r_h.at[pl.ds(seg * _NEG * _C, _NEG * _C)],
                            idx_n)
            cps = [
                pltpu.async_copy(U_h.at[idx_u], u_rows, sem),
                pltpu.async_copy(V_h.at[idx_v], v_rows, sem),
            ]
            for j in range(_NEG):
                cps.append(pltpu.async_copy(
                    V_h.at[idx_n.at[pl.ds(j * _C, _C)]],
                    n_rows.at[pl.ds(j * _C, _C)], sem))
            for cp in cps:
                cp.wait()

            def pos_body(g, _):
                acc = jnp.zeros((16,), jnp.float32)
                for kk in range(16):
                    b = g * 16 + kk
                    p = (u_rows[b, pl.ds(0, 16)] * v_rows[b, pl.ds(0, 16)]
                         + u_rows[b, pl.ds(16, 16)] * v_rows[b, pl.ds(16, 16)]
                         + u_rows[b, pl.ds(32, 16)] * v_rows[b, pl.ds(32, 16)]
                         + u_rows[b, pl.ds(48, 16)] * v_rows[b, pl.ds(48, 16)])
                    acc = jnp.where(li == kk, jnp.sum(p), acc)
                sc_pos[pl.ds(g * 16, 16)] = acc
                return 0

            lax.fori_loop(0, _C // 16, pos_body, 0)

            def neg_body(b, _):
                u0 = u_rows[b, pl.ds(0, 16)]
                u1 = u_rows[b, pl.ds(16, 16)]
                u2 = u_rows[b, pl.ds(32, 16)]
                u3 = u_rows[b, pl.ds(48, 16)]
                acc1 = jnp.zeros((16,), jnp.float32)
                acc2 = jnp.zeros((16,), jnp.float32)
                for j in range(_NEG):
                    r = j * _C + b
                    p = (n_rows[r, pl.ds(0, 16)] * u0
                         + n_rows[r, pl.ds(16, 16)] * u1
                         + n_rows[r, pl.ds(32, 16)] * u2
                         + n_rows[r, pl.ds(48, 16)] * u3)
                    s = -jnp.sum(p)
                    if j < 16:
                        acc1 = jnp.where(li == j, s, acc1)
                    else:
                        acc2 = jnp.where(li == (j - 16), s, acc2)
                sc_neg[pl.ds(b * _NEG, 16)] = acc1
                tail = sc_neg[pl.ds(b * _NEG + 16, 16)]
                sc_neg[pl.ds(b * _NEG + 16, 16)] = jnp.where(li < 4, acc2,
                                                             tail)
                return 0

            lax.fori_loop(0, _C, neg_body, 0)

            obase = seg * _SEG
            pltpu.sync_copy(sc_pos, out_h.at[pl.ds(obase, _C)])
            pltpu.sync_copy(sc_neg.at[pl.ds(0, _NEG * _C)],
                            out_h.at[pl.ds(obase + _C, _NEG * _C)])
            return 0

        lax.fori_loop(0, _NCH, chunk_body, 0)

    return k(pos_u, pos_v, neg_r, U, V)


def _tc_transpose(xt):
    d, n = xt.shape
    bn = 4096

    def body(x_ref, o_ref):
        x = x_ref[...]
        r = lax.broadcasted_iota(jnp.int32, (d, d), 0)
        c = lax.broadcasted_iota(jnp.int32, (d, d), 1)
        eye = (r == c).astype(jnp.float32)
        o_ref[...] = lax.dot_general(x, eye, (((0,), (0,)), ((), ())),
                                     preferred_element_type=jnp.float32)

    return pl.pallas_call(
        body,
        grid=(pl.cdiv(n, bn),),
        in_specs=[pl.BlockSpec((d, bn), lambda i: (0, i))],
        out_specs=pl.BlockSpec((bn, d), lambda i: (i, 0)),
        out_shape=jax.ShapeDtypeStruct((n, d), jnp.float32),
    )(xt)


def _tc_logsig_sum(x2d):
    def body(x_ref, o_ref):
        x = x_ref[...]
        ls = jnp.minimum(x, 0.0) - jnp.log1p(jnp.exp(-jnp.abs(x)))
        o_ref[0, 0] = -jnp.sum(ls)

    return pl.pallas_call(
        body,
        out_shape=jax.ShapeDtypeStruct((1, 1), jnp.float32),
        out_specs=pl.BlockSpec(memory_space=pltpu.SMEM),
    )(x2d)


def kernel(pos_u, pos_v, neg_v, U, V):
    neg_r = neg_v.reshape(_NW * _NCH, _C, _NEG).transpose(0, 2, 1).reshape(-1)
    U_rm = _tc_transpose(U.T)
    V_rm = _tc_transpose(V.T)
    scores = _sc_scores(pos_u, pos_v, neg_r, U_rm, V_rm)
    res = _tc_logsig_sum(scores.reshape(_NSCORE // 128, 128))
    return res[0, 0]


# speedup vs baseline: 2.0983x; 1.8268x over previous
import functools

import jax
import jax.numpy as jnp
from jax import lax
from jax.experimental import pallas as pl
from jax.experimental.pallas import tpu as pltpu
from jax.experimental.pallas import tpu_sc as plsc

_B = 16384
_D = 64
_NEG = 20
_NC = 2
_NS = 16
_NW = _NC * _NS
_BW = _B // _NW
_C = 32
_NCH = _BW // _C
_SEG = _C * (1 + _NEG)
_NSCORE = _B * (1 + _NEG)


def _sc_scores(pos_u, pos_v, neg_r, U, V):
    mesh = plsc.VectorSubcoreMesh(core_axis_name="c", subcore_axis_name="s")

    @functools.partial(
        pl.kernel,
        out_type=jax.ShapeDtypeStruct((_NSCORE,), jnp.float32),
        mesh=mesh,
        compiler_params=pltpu.CompilerParams(needs_layout_passes=False,
                                             use_tc_tiling_on_sc=False),
        scratch_types=[
            pltpu.VMEM((_C,), jnp.int32),
            pltpu.VMEM((_C,), jnp.int32),
            pltpu.VMEM((_NEG * _C,), jnp.int32),
            pltpu.VMEM((_C, 2 * _D), jnp.float32),
            pltpu.VMEM((_C, 2 * _D), jnp.float32),
            pltpu.VMEM((_NEG * _C, 2 * _D), jnp.float32),
            pltpu.VMEM((_C,), jnp.float32),
            pltpu.VMEM((_NEG * _C + 16,), jnp.float32),
            pltpu.SemaphoreType.DMA,
        ],
    )
    def k(pos_u_h, pos_v_h, neg_r_h, U_h, V_h, out_h,
          idx_u, idx_v, idx_n, u_rows, v_rows, n_rows, sc_pos, sc_neg, sem):
        wid = lax.axis_index("s") * _NC + lax.axis_index("c")
        li = lax.broadcasted_iota(jnp.int32, (16,), 0)

        def chunk_body(c, _):
            seg = wid * _NCH + c
            cbase = seg * _C
            pltpu.sync_copy(pos_u_h.at[pl.ds(cbase, _C)], idx_u)
            pltpu.sync_copy(pos_v_h.at[pl.ds(cbase, _C)], idx_v)
            pltpu.sync_copy(neg_r_h.at[pl.ds(seg * _NEG * _C, _NEG * _C)],
                            idx_n)
            cps = [
                pltpu.async_copy(U_h.at[idx_u], u_rows, sem),
                pltpu.async_copy(V_h.at[idx_v], v_rows, sem),
            ]
            for j in range(_NEG):
                cps.append(pltpu.async_copy(
                    V_h.at[idx_n.at[pl.ds(j * _C, _C)]],
                    n_rows.at[pl.ds(j * _C, _C)], sem))
            for cp in cps:
                cp.wait()

            def pos_body(g, _):
                acc = jnp.zeros((16,), jnp.float32)
                for kk in range(16):
                    b = g * 16 + kk
                    p = (u_rows[b, pl.ds(0, 16)] * v_rows[b, pl.ds(0, 16)]
                         + u_rows[b, pl.ds(16, 16)] * v_rows[b, pl.ds(16, 16)]
                         + u_rows[b, pl.ds(32, 16)] * v_rows[b, pl.ds(32, 16)]
                         + u_rows[b, pl.ds(48, 16)] * v_rows[b, pl.ds(48, 16)])
                    acc = jnp.where(li == kk, jnp.sum(p), acc)
                sc_pos[pl.ds(g * 16, 16)] = acc
                return 0

            lax.fori_loop(0, _C // 16, pos_body, 0)

            def neg_body(b, _):
                u0 = u_rows[b, pl.ds(0, 16)]
                u1 = u_rows[b, pl.ds(16, 16)]
                u2 = u_rows[b, pl.ds(32, 16)]
                u3 = u_rows[b, pl.ds(48, 16)]
                acc1 = jnp.zeros((16,), jnp.float32)
                acc2 = jnp.zeros((16,), jnp.float32)
                for j in range(_NEG):
                    r = j * _C + b
                    p = (n_rows[r, pl.ds(0, 16)] * u0
                         + n_rows[r, pl.ds(16, 16)] * u1
                         + n_rows[r, pl.ds(32, 16)] * u2
                         + n_rows[r, pl.ds(48, 16)] * u3)
                    s = -jnp.sum(p)
                    if j < 16:
                        acc1 = jnp.where(li == j, s, acc1)
                    else:
                        acc2 = jnp.where(li == (j - 16), s, acc2)
                sc_neg[pl.ds(b * _NEG, 16)] = acc1
                tail = sc_neg[pl.ds(b * _NEG + 16, 16)]
                sc_neg[pl.ds(b * _NEG + 16, 16)] = jnp.where(li < 4, acc2,
                                                             tail)
                return 0

            lax.fori_loop(0, _C, neg_body, 0)

            obase = seg * _SEG
            pltpu.sync_copy(sc_pos, out_h.at[pl.ds(obase, _C)])
            pltpu.sync_copy(sc_neg.at[pl.ds(0, _NEG * _C)],
                            out_h.at[pl.ds(obase + _C, _NEG * _C)])
            return 0

        lax.fori_loop(0, _NCH, chunk_body, 0)

    return k(pos_u, pos_v, neg_r, U, V)


def _tc_transpose(xt):
    d, n = xt.shape
    bn = 4096

    def body(x_ref, o_ref):
        x = x_ref[...]
        r = lax.broadcasted_iota(jnp.int32, (d, 2 * d), 0)
        c = lax.broadcasted_iota(jnp.int32, (d, 2 * d), 1)
        sel = (r == c).astype(jnp.float32)
        o_ref[...] = lax.dot_general(x, sel, (((0,), (0,)), ((), ())),
                                     preferred_element_type=jnp.float32)

    return pl.pallas_call(
        body,
        grid=(pl.cdiv(n, bn),),
        in_specs=[pl.BlockSpec((d, bn), lambda i: (0, i))],
        out_specs=pl.BlockSpec((bn, 2 * d), lambda i: (i, 0)),
        out_shape=jax.ShapeDtypeStruct((n, 2 * d), jnp.float32),
    )(xt)


def _tc_logsig_sum(x2d):
    def body(x_ref, o_ref):
        x = x_ref[...]
        ls = jnp.minimum(x, 0.0) - jnp.log1p(jnp.exp(-jnp.abs(x)))
        o_ref[0, 0] = -jnp.sum(ls)

    return pl.pallas_call(
        body,
        out_shape=jax.ShapeDtypeStruct((1, 1), jnp.float32),
        out_specs=pl.BlockSpec(memory_space=pltpu.SMEM),
    )(x2d)


def kernel(pos_u, pos_v, neg_v, U, V):
    neg_r = neg_v.reshape(_NW * _NCH, _C, _NEG).transpose(0, 2, 1).reshape(-1)
    U_rm = _tc_transpose(U.T)
    V_rm = _tc_transpose(V.T)
    scores = _sc_scores(pos_u, pos_v, neg_r, U_rm, V_rm)
    res = _tc_logsig_sum(scores.reshape(_NSCORE // 128, 128))
    return res[0, 0]


# speedup vs baseline: 2.5106x; 1.1964x over previous
import functools

import jax
import jax.numpy as jnp
from jax import lax
from jax.experimental import pallas as pl
from jax.experimental.pallas import tpu as pltpu
from jax.experimental.pallas import tpu_sc as plsc

_B = 16384
_D = 64
_NEG = 20
_NC = 2
_NS = 16
_NW = _NC * _NS
_BW = _B // _NW
_C = 32
_NCH = _BW // _C
_SEG = _C * (1 + _NEG)
_NSCORE = _B * (1 + _NEG)


def _sc_scores(pos_u, pos_v, neg_r, U, V):
    mesh = plsc.VectorSubcoreMesh(core_axis_name="c", subcore_axis_name="s")

    @functools.partial(
        pl.kernel,
        out_type=jax.ShapeDtypeStruct((_NSCORE,), jnp.float32),
        mesh=mesh,
        compiler_params=pltpu.CompilerParams(needs_layout_passes=False,
                                             use_tc_tiling_on_sc=False),
        scratch_types=[
            pltpu.VMEM((_C,), jnp.int32),
            pltpu.VMEM((_C,), jnp.int32),
            pltpu.VMEM((_NEG * _C,), jnp.int32),
            pltpu.VMEM((_C,), jnp.int32),
            pltpu.VMEM((_C,), jnp.int32),
            pltpu.VMEM((_NEG * _C,), jnp.int32),
            pltpu.VMEM((_C, 2 * _D), jnp.float32),
            pltpu.VMEM((_C, 2 * _D), jnp.float32),
            pltpu.VMEM((_NEG * _C, 2 * _D), jnp.float32),
            pltpu.VMEM((_C,), jnp.float32),
            pltpu.VMEM((_NEG * _C + 16,), jnp.float32),
            pltpu.SemaphoreType.DMA,
        ],
    )
    def k(pos_u_h, pos_v_h, neg_r_h, U_h, V_h, out_h,
          idx_u, idx_v, idx_n, idx_u_g, idx_v_g, idx_n_g,
          u_rows, v_rows, n_rows, sc_pos, sc_neg, sem):
        wid = lax.axis_index("s") * _NC + lax.axis_index("c")
        li = lax.broadcasted_iota(jnp.int32, (16,), 0)

        def _splat(x):
            return jnp.full((16,), x, jnp.int32)

        def _row(ref, r, pref):
            pv = ((plsc.load_gather(pref, [_splat(r)]) >> 11) & 1) * _D + li
            return [plsc.load_gather(ref, [_splat(r), pv + 16 * kq])
                    for kq in range(4)]

        def chunk_body(c, _):
            seg = wid * _NCH + c
            cbase = seg * _C
            pltpu.sync_copy(pos_u_h.at[pl.ds(cbase, _C)], idx_u)
            pltpu.sync_copy(pos_v_h.at[pl.ds(cbase, _C)], idx_v)
            pltpu.sync_copy(neg_r_h.at[pl.ds(seg * _NEG * _C, _NEG * _C)],
                            idx_n)
            def _line(v):
                return ((v >> 12) << 11) | (v & 2047)

            for i in range(_C // 16):
                idx_u_g[pl.ds(i * 16, 16)] = _line(idx_u[pl.ds(i * 16, 16)])
                idx_v_g[pl.ds(i * 16, 16)] = _line(idx_v[pl.ds(i * 16, 16)])

            def shift_body(i, _):
                idx_n_g[pl.ds(i * 16, 16)] = _line(idx_n[pl.ds(i * 16, 16)])
                return 0

            lax.fori_loop(0, _NEG * _C // 16, shift_body, 0)
            cps = [
                pltpu.async_copy(U_h.at[idx_u_g], u_rows, sem),
                pltpu.async_copy(V_h.at[idx_v_g], v_rows, sem),
            ]
            for j in range(_NEG):
                cps.append(pltpu.async_copy(
                    V_h.at[idx_n_g.at[pl.ds(j * _C, _C)]],
                    n_rows.at[pl.ds(j * _C, _C)], sem))
            for cp in cps:
                cp.wait()

            def pos_body(g, _):
                acc = jnp.zeros((16,), jnp.float32)
                for kk in range(16):
                    b = g * 16 + kk
                    uu = _row(u_rows, b, idx_u)
                    vv = _row(v_rows, b, idx_v)
                    p = (uu[0] * vv[0] + uu[1] * vv[1]
                         + uu[2] * vv[2] + uu[3] * vv[3])
                    acc = jnp.where(li == kk, jnp.sum(p), acc)
                sc_pos[pl.ds(g * 16, 16)] = acc
                return 0

            lax.fori_loop(0, _C // 16, pos_body, 0)

            def neg_body(b, _):
                u0, u1, u2, u3 = _row(u_rows, b, idx_u)
                acc1 = jnp.zeros((16,), jnp.float32)
                acc2 = jnp.zeros((16,), jnp.float32)
                for j in range(_NEG):
                    r = j * _C + b
                    nn = _row(n_rows, r, idx_n)
                    p = (nn[0] * u0 + nn[1] * u1
                         + nn[2] * u2 + nn[3] * u3)
                    s = -jnp.sum(p)
                    if j < 16:
                        acc1 = jnp.where(li == j, s, acc1)
                    else:
                        acc2 = jnp.where(li == (j - 16), s, acc2)
                sc_neg[pl.ds(b * _NEG, 16)] = acc1
                tail = sc_neg[pl.ds(b * _NEG + 16, 16)]
                sc_neg[pl.ds(b * _NEG + 16, 16)] = jnp.where(li < 4, acc2,
                                                             tail)
                return 0

            lax.fori_loop(0, _C, neg_body, 0)

            obase = seg * _SEG
            pltpu.sync_copy(sc_pos, out_h.at[pl.ds(obase, _C)])
            pltpu.sync_copy(sc_neg.at[pl.ds(0, _NEG * _C)],
                            out_h.at[pl.ds(obase + _C, _NEG * _C)])
            return 0

        lax.fori_loop(0, _NCH, chunk_body, 0)

    return k(pos_u, pos_v, neg_r, U, V)


def _tc_transpose(xt):
    d, n = xt.shape
    bn = 4096

    def body(x_ref, o_ref):
        x = x_ref[...]
        xab = jnp.concatenate([x[:, : bn // 2], x[:, bn // 2:]], axis=0)
        r = lax.broadcasted_iota(jnp.int32, (2 * d, 2 * d), 0)
        c = lax.broadcasted_iota(jnp.int32, (2 * d, 2 * d), 1)
        eye = (r == c).astype(jnp.float32)
        o_ref[...] = lax.dot_general(xab, eye, (((0,), (0,)), ((), ())),
                                     preferred_element_type=jnp.float32)

    nb = pl.cdiv(n, bn)
    return pl.pallas_call(
        body,
        grid=(nb,),
        in_specs=[pl.BlockSpec((d, bn), lambda i: (0, i))],
        out_specs=pl.BlockSpec((bn // 2, 2 * d), lambda i: (i, 0)),
        out_shape=jax.ShapeDtypeStruct((nb * bn // 2, 2 * d), jnp.float32),
    )(xt)


def _tc_logsig_sum(x2d):
    def body(x_ref, o_ref):
        x = x_ref[...]
        ls = jnp.minimum(x, 0.0) - jnp.log1p(jnp.exp(-jnp.abs(x)))
        o_ref[0, 0] = -jnp.sum(ls)

    return pl.pallas_call(
        body,
        out_shape=jax.ShapeDtypeStruct((1, 1), jnp.float32),
        out_specs=pl.BlockSpec(memory_space=pltpu.SMEM),
    )(x2d)


def kernel(pos_u, pos_v, neg_v, U, V):
    neg_r = neg_v.reshape(_NW * _NCH, _C, _NEG).transpose(0, 2, 1).reshape(-1)
    U_rm = _tc_transpose(U.T)
    V_rm = _tc_transpose(V.T)
    scores = _sc_scores(pos_u, pos_v, neg_r, U_rm, V_rm)
    res = _tc_logsig_sum(scores.reshape(_NSCORE // 128, 128))
    return res[0, 0]
